# baseline (speedup 1.0000x reference)
import functools

import jax
import jax.numpy as jnp
from jax import lax
from jax.experimental import pallas as pl
from jax.experimental.pallas import tpu as pltpu
from jax.experimental.pallas import tpu_sc as plsc


def _prep_body(t_ref, q_ref, lo_ref, w_ref):
    t_col = t_ref[...]
    q_row = q_ref[...]
    mask = t_col <= q_row
    cnt = jnp.sum(mask.astype(jnp.int32), axis=0, keepdims=True)
    tmax = t_col[-1:, :]
    tmin = t_col[:1, :]
    t0 = jnp.max(jnp.where(mask, t_col, tmin - 1.0), axis=0, keepdims=True)
    t1 = jnp.min(jnp.where(mask, tmax + 1.0, t_col), axis=0, keepdims=True)
    idx = jnp.clip(cnt, 1, t_ref.shape[0] - 1)
    lo_ref[...] = idx - 1
    w_ref[...] = (q_row - t0) / (t1 - t0)


def _prep(query_t, t):
    T = t.shape[0]
    S = query_t.shape[0]
    lo, w = pl.pallas_call(
        _prep_body,
        out_shape=(
            jax.ShapeDtypeStruct((1, S), jnp.int32),
            jax.ShapeDtypeStruct((1, S), jnp.float32),
        ),
    )(t.reshape(T, 1), query_t.reshape(1, S))
    return lo.reshape(S), w.reshape(S)


def _pack_body(x_ref, o_ref):
    x = x_ref[...]
    rb, d, b = x.shape
    u = lax.bitcast_convert_type(x.astype(jnp.bfloat16), jnp.uint16)
    u = u.astype(jnp.int32)
    ur = u.reshape(rb, d // 2, 2, b)
    w32 = ur[:, :, 0, :] | (ur[:, :, 1, :] << 16)
    wt = jnp.swapaxes(w32, 1, 2)
    wr = wt.reshape(rb, b // 4, 4, d // 2)
    hw = d // 2
    for m in range(4):
        o_ref[:, :, m * hw:(m + 1) * hw] = wr[:, :, m, :]


def _pack(x_t, rb):
    R, D, B = x_t.shape
    assert R % rb == 0
    return pl.pallas_call(
        _pack_body,
        grid=(R // rb,),
        in_specs=[pl.BlockSpec((rb, D, B), lambda i: (i, 0, 0))],
        out_specs=pl.BlockSpec((rb, B // 4, 2 * D), lambda i: (i, 0, 0)),
        out_shape=jax.ShapeDtypeStruct((R, B // 4, 2 * D), jnp.int32),
    )(x_t)


def _sc_spline(lo, w, x0q, xq, x1q, *, B, T, D, S):
    info = plsc.get_sparse_core_info()
    NC, NS = info.num_cores, info.num_subcores
    NW = NC * NS
    assert B % NW == 0
    nb = B // NW
    npair = nb // 2
    W = D
    WP = W + 1
    NDT = D // 8
    NST = S // 128

    mesh = plsc.VectorSubcoreMesh(core_axis_name="c", subcore_axis_name="s")

    @functools.partial(
        pl.kernel,
        out_type=jax.ShapeDtypeStruct((B, NDT, NST, 8, 128), jnp.float32),
        mesh=mesh,
        scratch_types=[
            pltpu.VMEM((2, T, WP), jnp.int32),
            pltpu.VMEM((NDT, NST, 8, 128), jnp.float32),
            pltpu.VMEM((S,), jnp.int32),
            pltpu.VMEM((S,), jnp.float32),
            pltpu.SemaphoreType.DMA((2, 3)),
            pltpu.SemaphoreType.DMA((2,)),
        ],
        compiler_params=pltpu.CompilerParams(
            use_tc_tiling_on_sc=False,
            needs_layout_passes=False,
        ),
    )
    def run(lo_hbm, w_hbm, x0_hbm, xq_hbm, x1_hbm, out_hbm,
            colp, outp, lo_v, w_v, sem_in, sem_out):
        wid = lax.axis_index("s") * NC + lax.axis_index("c")
        pltpu.sync_copy(lo_hbm, lo_v)
        pltpu.sync_copy(w_hbm, w_v)
        p0 = wid * npair

        def start_pair(jp, slot):
            r = p0 + jp
            rr = r // 2
            woff = (r % 2) * W
            c0 = pltpu.make_async_copy(
                x0_hbm.at[0, rr, pl.ds(woff, W)],
                colp.at[slot, 0, pl.ds(0, W)], sem_in.at[slot, 0])
            c1 = pltpu.make_async_copy(
                xq_hbm.at[:, rr, pl.ds(woff, W)],
                colp.at[slot, pl.ds(1, T - 2), pl.ds(0, W)],
                sem_in.at[slot, 1])
            c2 = pltpu.make_async_copy(
                x1_hbm.at[0, rr, pl.ds(woff, W)],
                colp.at[slot, T - 1, pl.ds(0, W)],
                sem_in.at[slot, 2])
            c0.start()
            c1.start()
            c2.start()
            return (c0, c1, c2)

        pending = start_pair(0, 0)
        out_handles = [None, None]

        for jp in range(npair):
            slot = jp % 2
            if jp + 1 < npair:
                nxt = start_pair(jp + 1, (jp + 1) % 2)
            for c in pending:
                c.wait()
            if jp + 1 < npair:
                pending = nxt
            col = colp.at[slot]

            for pcol in range(2):
                b = (p0 + jp) * 2 + pcol
                jbase = pcol * (W // 2)

                for h in range(2):
                    if out_handles[h] is not None:
                        out_handles[h].wait()

                    def s_loop(g, carry):
                        s0 = g * 16
                        st = g // 8
                        si0 = (g % 8) * 16
                        lo16 = lo_v[pl.ds(s0, 16)]
                        hi16 = lo16 + 1
                        w16 = w_v[pl.ds(s0, 16)]

                        def d_loop(dd, carry2):
                            vals = []
                            for m in range(4):
                                j16 = jnp.zeros((16,), jnp.int32) + (
                                    jbase + dd * 4 + m)
                                aw = plsc.load_gather(col, [lo16, j16])
                                cw = plsc.load_gather(col, [hi16, j16])
                                a0, a1 = plsc.unpack(
                                    plsc.bitcast(aw, jnp.bfloat16),
                                    format=plsc.PackFormat.INTERLEAVED)
                                c0, c1 = plsc.unpack(
                                    plsc.bitcast(cw, jnp.bfloat16),
                                    format=plsc.PackFormat.INTERLEAVED)
                                vals.append(a0 + w16 * (c0 - a0))
                                vals.append(a1 + w16 * (c1 - a1))
                            for m in range(4):
                                outp[dd, st, 2 * m, pl.ds(si0, 16)] = (
                                    vals[2 * m])
                                outp[dd, st, 2 * m + 1, pl.ds(si0, 16)] = (
                                    vals[2 * m + 1])
                            return carry2

                        lax.fori_loop(h * (NDT // 2), (h + 1) * (NDT // 2),
                                      d_loop, 0)
                        return carry

                    lax.fori_loop(0, S // 16, s_loop, 0)
                    oc = pltpu.make_async_copy(
                        outp.at[pl.ds(h * (NDT // 2), NDT // 2)],
                        out_hbm.at[b, pl.ds(h * (NDT // 2), NDT // 2)],
                        sem_out.at[h])
                    oc.start()
                    out_handles[h] = oc

        for h in range(2):
            if out_handles[h] is not None:
                out_handles[h].wait()

    return run(lo, w, x0q, xq, x1q)


def kernel(query_t, t, x0, knots, x1):
    T = t.shape[0]
    S = query_t.shape[0]
    B, D = knots.shape[1], knots.shape[2]
    lo, w = _prep(query_t, t)
    knots_t = jnp.transpose(knots, (0, 2, 1))
    x0_t = jnp.transpose(x0, (0, 2, 1))
    x1_t = jnp.transpose(x1, (0, 2, 1))
    xq = _pack(knots_t, 6)
    x0q = _pack(x0_t, 1)
    x1q = _pack(x1_t, 1)
    out5 = _sc_spline(lo, w, x0q, xq, x1q, B=B, T=T, D=D, S=S)
    return jnp.transpose(out5, (0, 2, 4, 1, 3)).reshape(B, S, D)


# speedup vs baseline: 1.0029x; 1.0029x over previous
import functools

import jax
import jax.numpy as jnp
from jax import lax
from jax.experimental import pallas as pl
from jax.experimental.pallas import tpu as pltpu
from jax.experimental.pallas import tpu_sc as plsc


def _prep_body(t_ref, q_ref, lo_ref, w_ref):
    t_col = t_ref[...]
    q_row = q_ref[...]
    mask = t_col <= q_row
    cnt = jnp.sum(mask.astype(jnp.int32), axis=0, keepdims=True)
    tmax = t_col[-1:, :]
    tmin = t_col[:1, :]
    t0 = jnp.max(jnp.where(mask, t_col, tmin - 1.0), axis=0, keepdims=True)
    t1 = jnp.min(jnp.where(mask, tmax + 1.0, t_col), axis=0, keepdims=True)
    idx = jnp.clip(cnt, 1, t_ref.shape[0] - 1)
    lo_ref[...] = idx - 1
    w_ref[...] = (q_row - t0) / (t1 - t0)


def _prep(query_t, t):
    T = t.shape[0]
    S = query_t.shape[0]
    lo, w = pl.pallas_call(
        _prep_body,
        out_shape=(
            jax.ShapeDtypeStruct((1, S), jnp.int32),
            jax.ShapeDtypeStruct((1, S), jnp.float32),
        ),
    )(t.reshape(T, 1), query_t.reshape(1, S))
    return lo.reshape(S), w.reshape(S)


def _sc_spline(lo, w, x0r, knotsr, x1r, *, B, T, D, S):
    info = plsc.get_sparse_core_info()
    NC, NS = info.num_cores, info.num_subcores
    NW = NC * NS
    assert B % NW == 0
    nb = B // NW
    DP = D + 1
    NDT = D // 8
    NST = S // 128

    mesh = plsc.VectorSubcoreMesh(core_axis_name="c", subcore_axis_name="s")

    @functools.partial(
        pl.kernel,
        out_type=jax.ShapeDtypeStruct((B, NDT, NST, 8, 128), jnp.float32),
        mesh=mesh,
        scratch_types=[
            pltpu.VMEM((2, T, DP), jnp.float32),
            pltpu.VMEM((NDT, NST, 8, 128), jnp.float32),
            pltpu.VMEM((S,), jnp.int32),
            pltpu.VMEM((S,), jnp.float32),
            pltpu.SemaphoreType.DMA((2, 3)),
            pltpu.SemaphoreType.DMA((2,)),
        ],
        compiler_params=pltpu.CompilerParams(
            use_tc_tiling_on_sc=False,
            needs_layout_passes=False,
        ),
    )
    def run(lo_hbm, w_hbm, x0_hbm, knots_hbm, x1_hbm, out_hbm,
            col2, outp, lo_v, w_v, sem_in, sem_out):
        wid = lax.axis_index("s") * NC + lax.axis_index("c")
        pltpu.sync_copy(lo_hbm, lo_v)
        pltpu.sync_copy(w_hbm, w_v)
        b0 = wid * nb

        def start_col(j, slot):
            b = b0 + j
            r = b // 2
            pcol = (b % 2) * D
            c0 = pltpu.make_async_copy(
                x0_hbm.at[0, r, pl.ds(pcol, D)],
                col2.at[slot, 0, pl.ds(0, D)], sem_in.at[slot, 0])
            c1 = pltpu.make_async_copy(
                knots_hbm.at[:, r, pl.ds(pcol, D)],
                col2.at[slot, pl.ds(1, T - 2), pl.ds(0, D)],
                sem_in.at[slot, 1])
            c2 = pltpu.make_async_copy(
                x1_hbm.at[0, r, pl.ds(pcol, D)],
                col2.at[slot, T - 1, pl.ds(0, D)],
                sem_in.at[slot, 2])
            c0.start()
            c1.start()
            c2.start()
            return (c0, c1, c2)

        pending = start_col(0, 0)
        out_handles = [None, None]

        for j in range(nb):
            slot = j % 2
            if j + 1 < nb:
                nxt = start_col(j + 1, (j + 1) % 2)
            for c in pending:
                c.wait()
            if j + 1 < nb:
                pending = nxt
            col = col2.at[slot]
            b = b0 + j

            for h in range(2):
                if out_handles[h] is not None:
                    out_handles[h].wait()

                def s_loop(g, carry):
                    s0 = g * 16
                    st = g // 8
                    si0 = (g % 8) * 16
                    lo16 = lo_v[pl.ds(s0, 16)]
                    hi16 = lo16 + 1
                    w16 = w_v[pl.ds(s0, 16)]

                    def d_loop(dd, carry2):
                        d0 = dd * 8
                        vals = []
                        for k in range(8):
                            d16 = jnp.zeros((16,), jnp.int32) + (d0 + k)
                            a = plsc.load_gather(col, [lo16, d16])
                            c = plsc.load_gather(col, [hi16, d16])
                            vals.append(a + w16 * (c - a))
                        for k in range(8):
                            outp[dd, st, k, pl.ds(si0, 16)] = vals[k]
                        return carry2

                    lax.fori_loop(h * (NDT // 2), (h + 1) * (NDT // 2), d_loop, 0)
                    return carry

                lax.fori_loop(0, S // 16, s_loop, 0)
                oc = pltpu.make_async_copy(
                    outp.at[pl.ds(h * (NDT // 2), NDT // 2)],
                    out_hbm.at[b, pl.ds(h * (NDT // 2), NDT // 2)],
                    sem_out.at[h])
                oc.start()
                out_handles[h] = oc

        for h in range(2):
            if out_handles[h] is not None:
                out_handles[h].wait()

    return run(lo, w, x0r, knotsr, x1r)


def kernel(query_t, t, x0, knots, x1):
    T = t.shape[0]
    S = query_t.shape[0]
    B, D = knots.shape[1], knots.shape[2]
    lo, w = _prep(query_t, t)
    knotsr = knots.reshape(T - 2, B // 2, 2 * D)
    x0r = x0.reshape(1, B // 2, 2 * D)
    x1r = x1.reshape(1, B // 2, 2 * D)
    out5 = _sc_spline(lo, w, x0r, knotsr, x1r, B=B, T=T, D=D, S=S)
    return jnp.transpose(out5, (0, 2, 4, 1, 3)).reshape(B, S, D)
